# baseline (speedup 1.0000x reference)
import jax
import jax.numpy as jnp
import numpy as np
from jax.experimental import pallas as pl

WS = 8
B, H, W = 16, 512, 512
GH, GW = H // WS, W // WS
CELL = WS * WS
PER_BATCH_CAT = GH * GW * CELL
PER_BATCH_BERN = GH * GW

K1 = (np.uint32(928981903), np.uint32(3453687069))
K2 = (np.uint32(4146024105), np.uint32(2718843009))
TINY = np.float32(np.finfo(np.float32).tiny)


def _rotl(x, d):
    return (x << np.uint32(d)) | (x >> np.uint32(32 - d))


def _threefry_bits(key, x1):
    k0, k1 = key
    ks = (k0, k1, np.uint32(np.uint32(k0) ^ np.uint32(k1) ^ np.uint32(0x1BD11BDA)))
    rot = ((13, 15, 26, 6), (17, 29, 16, 24))
    x0 = jnp.full_like(x1, ks[0])
    x1 = x1 + ks[1]
    for i in range(5):
        for r in rot[i % 2]:
            x0 = x0 + x1
            x1 = _rotl(x1, r) ^ x0
        x0 = x0 + ks[(i + 1) % 3]
        x1 = x1 + ks[(i + 2) % 3] + np.uint32(i + 1)
    return x0 ^ x1


def _u01(bits):
    f = jax.lax.bitcast_convert_type(
        (bits >> np.uint32(9)) | np.uint32(0x3F800000), jnp.float32)
    return f - jnp.float32(1.0)


def _log_sigmoid(x):
    return jnp.minimum(x, 0.0) - jnp.log1p(jnp.exp(-jnp.abs(x)))


def _pmax(a):
    s = jnp.max(a.reshape(GH, WS, W), axis=1)
    return jnp.max(s.reshape(GH, GW, WS), axis=2)


def _pmin(a):
    s = jnp.min(a.reshape(GH, WS, W), axis=1)
    return jnp.min(s.reshape(GH, GW, WS), axis=2)


def _psum(a):
    s = jnp.sum(a.reshape(GH, WS, W), axis=1)
    return jnp.sum(s.reshape(GH, GW, WS), axis=2)


def _pbc(a):
    return jnp.broadcast_to(a[:, None, :, None], (GH, WS, GW, WS)).reshape(H, W)


def _body(x_ref, lp_ref, acc_ref, xf_ref, yf_ref):
    b = pl.program_id(0).astype(jnp.uint32)
    img = x_ref[0]

    shp = (H, W)
    R = jax.lax.broadcasted_iota(jnp.uint32, shp, 0)
    C = jax.lax.broadcasted_iota(jnp.uint32, shp, 1)
    n = ((R >> np.uint32(3)) * np.uint32(GW * CELL)
         + (C >> np.uint32(3)) * np.uint32(CELL)
         + (R & np.uint32(7)) * np.uint32(WS)
         + (C & np.uint32(7))
         + b * np.uint32(PER_BATCH_CAT))
    u = _u01(_threefry_bits(K1, n)) + TINY
    score = img - jnp.log(-jnp.log(u))

    cid = ((R & np.uint32(7)) * np.uint32(WS) + (C & np.uint32(7))).astype(jnp.int32)
    mx = _pmax(score)
    choice = _pmin(jnp.where(score == _pbc(mx), cid, CELL))
    selected = _psum(jnp.where(cid == _pbc(choice), img, 0.0))
    xmax = _pmax(img)
    sumexp = _psum(jnp.exp(img - _pbc(xmax)))
    logp_cat = (selected - xmax) - jnp.log(sumexp)

    shp2 = (GH, GW)
    n2 = (jax.lax.broadcasted_iota(jnp.uint32, shp2, 0) * np.uint32(GW)
          + jax.lax.broadcasted_iota(jnp.uint32, shp2, 1)
          + b * np.uint32(PER_BATCH_BERN))
    u2 = _u01(_threefry_bits(K2, n2))
    p = jax.nn.sigmoid(selected)
    acc = (u2 < p).astype(jnp.float32)

    logp_bern = acc * _log_sigmoid(selected) + (1.0 - acc) * _log_sigmoid(-selected)
    lp_ref[0] = logp_cat + logp_bern
    acc_ref[0] = acc

    gi = jax.lax.broadcasted_iota(jnp.int32, shp2, 0)
    gj = jax.lax.broadcasted_iota(jnp.int32, shp2, 1)
    xf_ref[0] = (gj * WS + (choice & 7)).astype(jnp.float32)
    yf_ref[0] = (gi * WS + (choice >> 3)).astype(jnp.float32)


def _run(x, interpret=False):
    xr = x.reshape(B, H, W)
    out = jax.ShapeDtypeStruct((B, GH, GW), jnp.float32)
    ospec = pl.BlockSpec((1, GH, GW), lambda b: (b, 0, 0))
    lp, acc, xf, yf = pl.pallas_call(
        _body,
        grid=(B,),
        in_specs=[pl.BlockSpec((1, H, W), lambda b: (b, 0, 0))],
        out_specs=[ospec, ospec, ospec, ospec],
        out_shape=[out, out, out, out],
        interpret=interpret,
    )(xr)
    xy = jnp.stack([xf, yf], axis=-1)
    return xy, lp, acc > 0


def kernel(x):
    return _run(x)


# speedup vs baseline: 1.5826x; 1.5826x over previous
import jax
import jax.numpy as jnp
import numpy as np
from jax.experimental import pallas as pl

WS = 8
B, H, W = 16, 512, 512
GH, GW = H // WS, W // WS
CELL = WS * WS
PER_BATCH_CAT = GH * GW * CELL
PER_BATCH_BERN = GH * GW

K1 = (np.uint32(928981903), np.uint32(3453687069))
K2 = (np.uint32(4146024105), np.uint32(2718843009))
TINY = np.float32(np.finfo(np.float32).tiny)


def _rotl(x, d):
    return (x << np.uint32(d)) | (x >> np.uint32(32 - d))


def _threefry_bits(key, x1):
    k0, k1 = key
    ks = (k0, k1, np.uint32(np.uint32(k0) ^ np.uint32(k1) ^ np.uint32(0x1BD11BDA)))
    rot = ((13, 15, 26, 6), (17, 29, 16, 24))
    x0 = jnp.full_like(x1, ks[0])
    x1 = x1 + ks[1]
    for i in range(5):
        for r in rot[i % 2]:
            x0 = x0 + x1
            x1 = _rotl(x1, r) ^ x0
        x0 = x0 + ks[(i + 1) % 3]
        x1 = x1 + ks[(i + 2) % 3] + np.uint32(i + 1)
    return x0 ^ x1


def _u01(bits):
    f = jax.lax.bitcast_convert_type(
        (bits >> np.uint32(9)) | np.uint32(0x3F800000), jnp.float32)
    return f - jnp.float32(1.0)


def _log_sigmoid(x):
    return jnp.minimum(x, 0.0) - jnp.log1p(jnp.exp(-jnp.abs(x)))


def _lg_reduce(a, op):
    return op(a.reshape(GH, GW, WS), axis=2)


def _lg_bcast(a):
    return jnp.broadcast_to(a[:, :, None], (GH, GW, WS)).reshape(GH, W)


def _body(x_ref, lp_ref, acc_ref, xf_ref, yf_ref):
    b = pl.program_id(0).astype(jnp.uint32)
    img = x_ref[0]

    shp = (H, W)
    R = jax.lax.broadcasted_iota(jnp.uint32, shp, 0)
    C = jax.lax.broadcasted_iota(jnp.uint32, shp, 1)
    n = ((R >> np.uint32(3)) * np.uint32(GW * CELL)
         + (C >> np.uint32(3)) * np.uint32(CELL)
         + (R & np.uint32(7)) * np.uint32(WS)
         + (C & np.uint32(7))
         + b * np.uint32(PER_BATCH_CAT))
    u = _u01(_threefry_bits(K1, n)) + TINY
    score = img - jnp.log(-jnp.log(u))

    cube = score.reshape(GH, WS, W)
    m1 = jnp.max(cube, axis=1)
    mx = _lg_reduce(m1, jnp.max)

    c_cube = (jax.lax.broadcasted_iota(jnp.int32, (GH, WS, W), 1) * WS
              + (jax.lax.broadcasted_iota(jnp.int32, (GH, WS, W), 2) & 7))
    eq = cube == _lg_bcast(mx)[:, None, :]
    cmin1 = jnp.min(jnp.where(eq, c_cube, CELL), axis=1)
    choice = _lg_reduce(cmin1, jnp.min)

    img_cube = img.reshape(GH, WS, W)
    eq2 = c_cube == _lg_bcast(choice)[:, None, :]
    s1 = jnp.sum(jnp.where(eq2, img_cube, 0.0), axis=1)
    selected = _lg_reduce(s1, jnp.sum)

    e1 = jnp.sum(jnp.exp(img_cube), axis=1)
    logp_cat = selected - jnp.log(_lg_reduce(e1, jnp.sum))

    shp2 = (GH, GW)
    n2 = (jax.lax.broadcasted_iota(jnp.uint32, shp2, 0) * np.uint32(GW)
          + jax.lax.broadcasted_iota(jnp.uint32, shp2, 1)
          + b * np.uint32(PER_BATCH_BERN))
    u2 = _u01(_threefry_bits(K2, n2))
    p = jax.nn.sigmoid(selected)
    acc = (u2 < p).astype(jnp.float32)

    logp_bern = acc * _log_sigmoid(selected) + (1.0 - acc) * _log_sigmoid(-selected)
    lp_ref[0] = logp_cat + logp_bern
    acc_ref[0] = acc

    gi = jax.lax.broadcasted_iota(jnp.int32, shp2, 0)
    gj = jax.lax.broadcasted_iota(jnp.int32, shp2, 1)
    xf_ref[0] = (gj * WS + (choice & 7)).astype(jnp.float32)
    yf_ref[0] = (gi * WS + (choice >> 3)).astype(jnp.float32)


def _run(x, interpret=False):
    xr = x.reshape(B, H, W)
    out = jax.ShapeDtypeStruct((B, GH, GW), jnp.float32)
    ospec = pl.BlockSpec((1, GH, GW), lambda b: (b, 0, 0))
    lp, acc, xf, yf = pl.pallas_call(
        _body,
        grid=(B,),
        in_specs=[pl.BlockSpec((1, H, W), lambda b: (b, 0, 0))],
        out_specs=[ospec, ospec, ospec, ospec],
        out_shape=[out, out, out, out],
        interpret=interpret,
    )(xr)
    xy = jnp.stack([xf, yf], axis=-1)
    return xy, lp, acc > 0


def kernel(x):
    return _run(x)


# speedup vs baseline: 4.1219x; 2.6045x over previous
import jax
import jax.numpy as jnp
import numpy as np
from jax.experimental import pallas as pl

WS = 8
B, H, W = 16, 512, 512
GH, GW = H // WS, W // WS
NCELL = GH * GW
CELL = WS * WS
PER_BATCH_CAT = NCELL * CELL

K1 = (np.uint32(928981903), np.uint32(3453687069))
K2 = (np.uint32(4146024105), np.uint32(2718843009))
TINY = np.float32(np.finfo(np.float32).tiny)


def _rotl(x, d):
    return (x << np.uint32(d)) | (x >> np.uint32(32 - d))


def _threefry_bits(key, x1):
    k0, k1 = key
    ks = (k0, k1, np.uint32(np.uint32(k0) ^ np.uint32(k1) ^ np.uint32(0x1BD11BDA)))
    rot = ((13, 15, 26, 6), (17, 29, 16, 24))
    x0 = jnp.full_like(x1, ks[0])
    x1 = x1 + ks[1]
    for i in range(5):
        for r in rot[i % 2]:
            x0 = x0 + x1
            x1 = _rotl(x1, r) ^ x0
        x0 = x0 + ks[(i + 1) % 3]
        x1 = x1 + ks[(i + 2) % 3] + np.uint32(i + 1)
    return x0 ^ x1


def _u01(bits):
    f = jax.lax.bitcast_convert_type(
        (bits >> np.uint32(9)) | np.uint32(0x3F800000), jnp.float32)
    return f - jnp.float32(1.0)


def _log_sigmoid(x):
    return jnp.minimum(x, 0.0) - jnp.log1p(jnp.exp(-jnp.abs(x)))


def _body(ct_ref, lp_ref, acc_ref, xf_ref, yf_ref):
    b = pl.program_id(0).astype(jnp.uint32)
    a = ct_ref[0]

    shp = (CELL, NCELL)
    ci = jax.lax.broadcasted_iota(jnp.uint32, shp, 0)
    cell = jax.lax.broadcasted_iota(jnp.uint32, shp, 1)
    n = cell * np.uint32(CELL) + ci + b * np.uint32(PER_BATCH_CAT)
    u = _u01(_threefry_bits(K1, n)) + TINY
    score = a - jnp.log(-jnp.log(u))

    mx = jnp.max(score, axis=0, keepdims=True)
    lanes = ci.astype(jnp.int32)
    choice = jnp.min(jnp.where(score == mx, lanes, CELL), axis=0, keepdims=True)
    chm = lanes == choice

    selected = jnp.sum(jnp.where(chm, a, 0.0), axis=0, keepdims=True)
    xmax = jnp.max(a, axis=0, keepdims=True)
    sumexp = jnp.sum(jnp.exp(a - xmax), axis=0, keepdims=True)
    logp_cat = (selected - xmax) - jnp.log(sumexp)

    n2 = (jax.lax.broadcasted_iota(jnp.uint32, (1, NCELL), 1)
          + b * np.uint32(NCELL))
    u2 = _u01(_threefry_bits(K2, n2))
    p = jax.nn.sigmoid(selected)
    acc = (u2 < p).astype(jnp.float32)

    logp_bern = acc * _log_sigmoid(selected) + (1.0 - acc) * _log_sigmoid(-selected)
    lp_ref[0] = logp_cat + logp_bern
    acc_ref[0] = acc

    cell_row = jax.lax.broadcasted_iota(jnp.int32, (1, NCELL), 1)
    xf_ref[0] = ((cell_row & 63) * WS + (choice & 7)).astype(jnp.float32)
    yf_ref[0] = ((cell_row >> 6) * WS + (choice >> 3)).astype(jnp.float32)


def _run(x, interpret=False):
    ct = jnp.transpose(
        x.reshape(B, GH, WS, GW, WS), (0, 2, 4, 1, 3)).reshape(B, CELL, NCELL)
    out = jax.ShapeDtypeStruct((B, 1, NCELL), jnp.float32)
    ospec = pl.BlockSpec((1, 1, NCELL), lambda b: (b, 0, 0))
    lp, acc, xf, yf = pl.pallas_call(
        _body,
        grid=(B,),
        in_specs=[pl.BlockSpec((1, CELL, NCELL), lambda b: (b, 0, 0))],
        out_specs=[ospec, ospec, ospec, ospec],
        out_shape=[out, out, out, out],
        interpret=interpret,
    )(ct)
    lp = lp.reshape(B, GH, GW)
    acc = acc.reshape(B, GH, GW)
    xy = jnp.stack([xf.reshape(B, GH, GW), yf.reshape(B, GH, GW)], axis=-1)
    return xy, lp, acc > 0


def kernel(x):
    return _run(x)


# speedup vs baseline: 4.4212x; 1.0726x over previous
import jax
import jax.numpy as jnp
import numpy as np
from jax.experimental import pallas as pl

WS = 8
B, H, W = 16, 512, 512
GH, GW = H // WS, W // WS
NCELL = GH * GW
CELL = WS * WS
PER_BATCH_CAT = NCELL * CELL

K1 = (np.uint32(928981903), np.uint32(3453687069))
K2 = (np.uint32(4146024105), np.uint32(2718843009))
TINY = np.float32(np.finfo(np.float32).tiny)


def _rotl(x, d):
    return (x << np.uint32(d)) | (x >> np.uint32(32 - d))


def _threefry_bits(key, x1):
    k0, k1 = key
    ks = (k0, k1, np.uint32(np.uint32(k0) ^ np.uint32(k1) ^ np.uint32(0x1BD11BDA)))
    rot = ((13, 15, 26, 6), (17, 29, 16, 24))
    x0 = jnp.full_like(x1, ks[0])
    x1 = x1 + ks[1]
    for i in range(5):
        for r in rot[i % 2]:
            x0 = x0 + x1
            x1 = _rotl(x1, r) ^ x0
        x0 = x0 + ks[(i + 1) % 3]
        x1 = x1 + ks[(i + 2) % 3] + np.uint32(i + 1)
    return x0 ^ x1


def _u01(bits):
    f = jax.lax.bitcast_convert_type(
        (bits >> np.uint32(9)) | np.uint32(0x3F800000), jnp.float32)
    return f - jnp.float32(1.0)


def _log_sigmoid(x):
    return jnp.minimum(x, 0.0) - jnp.log1p(jnp.exp(-jnp.abs(x)))


CH, CW = 8, 512


def _body(ct_ref, lp_ref, acc_ref, xf_ref, yf_ref):
    b = pl.program_id(0).astype(jnp.uint32)
    a = ct_ref[0]

    shp = (CELL, CH, CW)
    ci = jax.lax.broadcasted_iota(jnp.uint32, shp, 0)
    cell = (jax.lax.broadcasted_iota(jnp.uint32, shp, 1) * np.uint32(CW)
            + jax.lax.broadcasted_iota(jnp.uint32, shp, 2))
    n = cell * np.uint32(CELL) + ci + b * np.uint32(PER_BATCH_CAT)
    u = _u01(_threefry_bits(K1, n)) + TINY
    score = a - jnp.log(-jnp.log(u))

    mx = jnp.max(score, axis=0)
    lanes = ci.astype(jnp.int32)
    choice = jnp.min(jnp.where(score == mx[None], lanes, CELL), axis=0)
    chm = lanes == choice[None]

    selected = jnp.sum(jnp.where(chm, a, 0.0), axis=0)
    xmax = jnp.max(a, axis=0)
    sumexp = jnp.sum(jnp.exp(a - xmax[None]), axis=0)
    logp_cat = (selected - xmax) - jnp.log(sumexp)

    shp2 = (CH, CW)
    cell2 = (jax.lax.broadcasted_iota(jnp.uint32, shp2, 0) * np.uint32(CW)
             + jax.lax.broadcasted_iota(jnp.uint32, shp2, 1))
    u2 = _u01(_threefry_bits(K2, cell2 + b * np.uint32(NCELL)))
    p = jax.nn.sigmoid(selected)
    acc = (u2 < p).astype(jnp.float32)

    logp_bern = acc * _log_sigmoid(selected) + (1.0 - acc) * _log_sigmoid(-selected)
    lp_ref[0] = logp_cat + logp_bern
    acc_ref[0] = acc

    celli = cell2.astype(jnp.int32)
    xf_ref[0] = ((celli & 63) * WS + (choice & 7)).astype(jnp.float32)
    yf_ref[0] = ((celli >> 6) * WS + (choice >> 3)).astype(jnp.float32)


def _run(x, interpret=False):
    ct = jnp.transpose(
        x.reshape(B, GH, WS, GW, WS), (0, 2, 4, 1, 3)).reshape(B, CELL, CH, CW)
    out = jax.ShapeDtypeStruct((B, CH, CW), jnp.float32)
    ospec = pl.BlockSpec((1, CH, CW), lambda b: (b, 0, 0))
    lp, acc, xf, yf = pl.pallas_call(
        _body,
        grid=(B,),
        in_specs=[pl.BlockSpec((1, CELL, CH, CW), lambda b: (b, 0, 0, 0))],
        out_specs=[ospec, ospec, ospec, ospec],
        out_shape=[out, out, out, out],
        interpret=interpret,
    )(ct)
    lp = lp.reshape(B, GH, GW)
    acc = acc.reshape(B, GH, GW)
    xy = jnp.stack([xf.reshape(B, GH, GW), yf.reshape(B, GH, GW)], axis=-1)
    return xy, lp, acc > 0


def kernel(x):
    return _run(x)
